# baseline (speedup 1.0000x reference)
import jax
import jax.numpy as jnp
from jax import lax
from jax.experimental import pallas as pl
from jax.experimental.pallas import tpu as pltpu
from jax.experimental.pallas import tpu_sc as plsc

N_NODES = 100000
D = 512
M = 2048
B = 8192
S = 16

NC = 2
NS = 16
NW = NC * NS
BPW = B // NW
CH = 4
ROWS = CH * S + CH
RPAD = 72
NCHW = BPW // CH
NBUF = 2
NOUTER = NCHW // NBUF
NCHT = B // CH
LANES = D // 16


def _sc_body(idx_hbm, feat_hbm, comb_hbm, idx_v, rows_v, stage_v,
             sem_g0, sem_g1, sem_o0, sem_o1):
    sem_g = (sem_g0, sem_g1)
    sem_o = (sem_o0, sem_o1)
    cid = lax.axis_index("c")
    sid = lax.axis_index("s")
    wid = cid * NS + sid
    c0 = wid * NCHW

    pltpu.sync_copy(idx_hbm.at[pl.ds(c0, NCHW)], idx_v)

    def start_gather(c, b):
        pltpu.async_copy(feat_hbm.at[idx_v.at[c]], rows_v.at[b], sem_g[b])

    def wait_gather(c, b):
        pltpu.make_async_copy(feat_hbm.at[idx_v.at[c]], rows_v.at[b],
                              sem_g[b]).wait()

    def out_slice(c):
        return comb_hbm.at[pl.ds((c0 + c) * CH, CH)]

    def start_out(c, b):
        pltpu.async_copy(stage_v.at[b], out_slice(c), sem_o[b])

    def wait_out(c, b):
        pltpu.make_async_copy(stage_v.at[b], out_slice(c), sem_o[b]).wait()

    scale = jnp.float32(1.0 / S)

    def accumulate(b):
        def jbody(j, carry):
            col = pl.ds(j * 16, 16)
            ncol = pl.ds(D + j * 16, 16)
            for v in range(CH):
                terms = [rows_v[b, v * S + s, col] for s in range(S)]
                while len(terms) > 1:
                    terms = [terms[2 * i] + terms[2 * i + 1]
                             for i in range(len(terms) // 2)]
                stage_v[b, v, ncol] = terms[0] * scale
                stage_v[b, v, col] = rows_v[b, CH * S + v, col]
            return carry

        lax.fori_loop(0, LANES, jbody, 0)

    for b in range(NBUF):
        start_gather(b, b)

    def outer(t, carry):
        for b in range(NBUF):
            c = t * NBUF + b
            wait_gather(c, b)

            @pl.when(t > 0)
            def _():
                wait_out(c - NBUF, b)

            accumulate(b)
            start_out(c, b)

            @pl.when(t < NOUTER - 1)
            def _():
                start_gather(c + NBUF, b)

        return carry

    lax.fori_loop(0, NOUTER, outer, 0)

    for b in range(NBUF):
        wait_out(NCHW - NBUF + b, b)


def _sc_gather(idx_all, features):
    mesh = plsc.VectorSubcoreMesh(core_axis_name="c", subcore_axis_name="s",
                                  num_cores=NC, num_subcores=NS)
    f = pl.kernel(
        _sc_body,
        out_type=jax.ShapeDtypeStruct((B, 2 * D), jnp.float32),
        mesh=mesh,
        scratch_types=[
            pltpu.VMEM((NCHW, RPAD), jnp.int32),
            pltpu.VMEM((NBUF, RPAD, D), jnp.float32),
            pltpu.VMEM((NBUF, CH, 2 * D), jnp.float32),
            pltpu.SemaphoreType.DMA,
            pltpu.SemaphoreType.DMA,
            pltpu.SemaphoreType.DMA,
            pltpu.SemaphoreType.DMA,
        ],
    )
    return f(idx_all, features)


BN = 1024


def _tc_body(w_ref, comb_ref, out_ref):
    cb = comb_ref[...]
    cb = jnp.where(jnp.isnan(cb), jnp.float32(0.0), cb)
    acc = lax.dot_general(w_ref[...], cb, (((1,), (1,)), ((), ())),
                          preferred_element_type=jnp.float32)
    out_ref[...] = jnp.maximum(acc, jnp.float32(0.0))


def _tc_matmul(W1, comb):
    return pl.pallas_call(
        _tc_body,
        grid=(B // BN,),
        in_specs=[
            pl.BlockSpec((M, 2 * D), lambda n: (0, 0)),
            pl.BlockSpec((BN, 2 * D), lambda n: (n, 0)),
        ],
        out_specs=pl.BlockSpec((M, BN), lambda n: (0, n)),
        out_shape=jax.ShapeDtypeStruct((M, B), jnp.float32),
    )(W1, comb)


def _pack_indices(nodes, neigh_idx):
    nidx_c = neigh_idx.reshape(NCHT, CH * S)
    nodes_c = nodes.reshape(NCHT, CH)
    pad = jnp.zeros((NCHT, RPAD - ROWS), jnp.int32)
    return jnp.concatenate([nidx_c, nodes_c, pad], axis=1)


def kernel(nodes, neigh_idx, features, W1):
    idx_all = _pack_indices(nodes, neigh_idx)
    comb = _sc_gather(idx_all, features)
    return _tc_matmul(W1, comb)
